# Initial kernel scaffold; baseline (speedup 1.0000x reference)
import jax
import jax.numpy as jnp
from jax.experimental import pallas as pl


def kernel(x, table):
    raise NotImplementedError("write your pallas kernel here")


# speedup vs baseline: 1.1028x; 1.1028x over previous
import functools

import jax
import jax.numpy as jnp
from jax import lax
from jax.experimental import pallas as pl
from jax.experimental.pallas import tpu as pltpu
from jax.experimental.pallas import tpu_sc as plsc

NUM_EMBEDDINGS = 1000000
EMBEDDING_DIM = 32
BATCH = 16384
HIST = 50

B = BATCH * HIST
NC, NS = 2, 16
NW = NC * NS
B_PER_W = B // NW
CHUNK = 1600
N_CHUNKS = B_PER_W // CHUNK


def _gather_body(table_hbm, idx_hbm, out_hbm, idx_v, rows_v, sem_g):
    wid = lax.axis_index("s") * NC + lax.axis_index("c")
    base = wid * B_PER_W
    for i in range(N_CHUNKS):
        off = base + i * CHUNK
        pltpu.sync_copy(idx_hbm.at[pl.ds(off, CHUNK)], idx_v)
        pltpu.async_copy(table_hbm.at[idx_v], rows_v, sem_g).wait()
        pltpu.sync_copy(rows_v, out_hbm.at[pl.ds(off, CHUNK)])


_sc_gather = pl.kernel(
    _gather_body,
    out_type=jax.ShapeDtypeStruct((B, EMBEDDING_DIM), jnp.float32),
    mesh=plsc.VectorSubcoreMesh(core_axis_name="c", subcore_axis_name="s"),
    scratch_types=[
        pltpu.VMEM((CHUNK,), jnp.int32),
        pltpu.VMEM((CHUNK, EMBEDDING_DIM), jnp.float32),
        pltpu.SemaphoreType.DMA,
    ],
    compiler_params=pltpu.CompilerParams(use_tc_tiling_on_sc=False),
)


@jax.jit
def kernel(x, table):
    idx = x.reshape(-1).astype(jnp.int32)
    out = _sc_gather(table, idx)
    return out.reshape(BATCH, HIST, EMBEDDING_DIM)


# speedup vs baseline: 1.1123x; 1.0086x over previous
import functools

import jax
import jax.numpy as jnp
from jax import lax
from jax.experimental import pallas as pl
from jax.experimental.pallas import tpu as pltpu
from jax.experimental.pallas import tpu_sc as plsc

NUM_EMBEDDINGS = 1000000
EMBEDDING_DIM = 32
BATCH = 16384
HIST = 50

B = BATCH * HIST
NC, NS = 2, 16
NW = NC * NS
B_PER_W = B // NW
CHUNK = 1600
N_CHUNKS = B_PER_W // CHUNK


def _gather_body(table_hbm, idx_hbm, out_hbm,
                 idx_v0, idx_v1, rows_v0, rows_v1,
                 sem_i0, sem_i1, sem_g0, sem_g1, sem_s0, sem_s1):
    wid = lax.axis_index("s") * NC + lax.axis_index("c")
    base = wid * B_PER_W
    idx_v = (idx_v0, idx_v1)
    rows_v = (rows_v0, rows_v1)
    sem_i = (sem_i0, sem_i1)
    sem_g = (sem_g0, sem_g1)
    sem_s = (sem_s0, sem_s1)

    def idx_load(i):
        return pltpu.async_copy(
            idx_hbm.at[pl.ds(base + i * CHUNK, CHUNK)], idx_v[i % 2], sem_i[i % 2])

    def gather(i):
        return pltpu.async_copy(table_hbm.at[idx_v[i % 2]], rows_v[i % 2], sem_g[i % 2])

    def store(i):
        return pltpu.async_copy(
            rows_v[i % 2], out_hbm.at[pl.ds(base + i * CHUNK, CHUNK)], sem_s[i % 2])

    loads = {0: idx_load(0), 1: idx_load(1)}
    loads[0].wait()
    gathers = {0: gather(0)}
    stores = {}
    for i in range(N_CHUNKS):
        nxt = i + 1
        if nxt < N_CHUNKS:
            loads[nxt].wait()
            if nxt - 2 >= 0:
                stores[nxt - 2].wait()
            gathers[nxt] = gather(nxt)
        gathers[i].wait()
        stores[i] = store(i)
        if i + 2 < N_CHUNKS:
            loads[i + 2] = idx_load(i + 2)
    stores[N_CHUNKS - 2].wait()
    stores[N_CHUNKS - 1].wait()


_sc_gather = pl.kernel(
    _gather_body,
    out_type=jax.ShapeDtypeStruct((B, EMBEDDING_DIM), jnp.float32),
    mesh=plsc.VectorSubcoreMesh(core_axis_name="c", subcore_axis_name="s"),
    scratch_types=[
        pltpu.VMEM((CHUNK,), jnp.int32),
        pltpu.VMEM((CHUNK,), jnp.int32),
        pltpu.VMEM((CHUNK, EMBEDDING_DIM), jnp.float32),
        pltpu.VMEM((CHUNK, EMBEDDING_DIM), jnp.float32),
        pltpu.SemaphoreType.DMA,
        pltpu.SemaphoreType.DMA,
        pltpu.SemaphoreType.DMA,
        pltpu.SemaphoreType.DMA,
        pltpu.SemaphoreType.DMA,
        pltpu.SemaphoreType.DMA,
    ],
    compiler_params=pltpu.CompilerParams(use_tc_tiling_on_sc=False),
)


@jax.jit
def kernel(x, table):
    idx = x.reshape(-1).astype(jnp.int32)
    out = _sc_gather(table, idx)
    return out.reshape(BATCH, HIST, EMBEDDING_DIM)


# speedup vs baseline: 1.4455x; 1.2995x over previous
import jax
import jax.numpy as jnp
from jax import lax
from jax.experimental import pallas as pl
from jax.experimental.pallas import tpu as pltpu
from jax.experimental.pallas import tpu_sc as plsc

NUM_EMBEDDINGS = 1000000
EMBEDDING_DIM = 32
BATCH = 16384
HIST = 50

NC, NS, L = 2, 16, 16
NW = NC * NS
BW = BATCH // NW
N_CHUNKS = HIST


def _transpose_chunk(rows, tbuf):
    def jb_body(jb, carry):
        row_idx = jb * L + lax.iota(jnp.int32, L)
        for d in range(EMBEDDING_DIM):
            col_idx = jnp.full((L,), d, jnp.int32)
            tbuf[d, pl.ds(jb * L, L)] = plsc.load_gather(rows, [row_idx, col_idx])
        return carry

    lax.fori_loop(0, BW // L, jb_body, 0, unroll=False)


def _gather_body(table_hbm, xt_hbm, out_hbm,
                 idx_all, rows_v0, rows_v1, tbuf0, tbuf1,
                 sem_i, sem_g0, sem_g1, sem_s0, sem_s1):
    wid = lax.axis_index("s") * NC + lax.axis_index("c")
    b0 = wid * BW
    rows_v = (rows_v0, rows_v1)
    tbuf = (tbuf0, tbuf1)
    sem_g = (sem_g0, sem_g1)
    sem_s = (sem_s0, sem_s1)

    pltpu.async_copy(xt_hbm.at[:, pl.ds(b0, BW)], idx_all, sem_i).wait()

    def gather(h, buf):
        return pltpu.async_copy(table_hbm.at[idx_all.at[h]], rows_v[buf], sem_g[buf])

    def store_desc(h, buf):
        return pltpu.make_async_copy(
            tbuf[buf], out_hbm.at[h, :, pl.ds(b0, BW)], sem_s[buf])

    def pair_body(k, carry):
        ha = 2 * k
        hb = 2 * k + 1
        ga = gather(ha, 0)
        gb = gather(hb, 1)

        @pl.when(k > 0)
        def _():
            store_desc(2 * k - 2, 0).wait()
            store_desc(2 * k - 1, 1).wait()

        ga.wait()
        _transpose_chunk(rows_v[0], tbuf[0])
        store_desc(ha, 0).start()
        gb.wait()
        _transpose_chunk(rows_v[1], tbuf[1])
        store_desc(hb, 1).start()
        return carry

    lax.fori_loop(0, N_CHUNKS // 2, pair_body, 0, unroll=False)
    store_desc(N_CHUNKS - 2, 0).wait()
    store_desc(N_CHUNKS - 1, 1).wait()


_sc_gather = pl.kernel(
    _gather_body,
    out_type=jax.ShapeDtypeStruct((HIST, EMBEDDING_DIM, BATCH), jnp.float32),
    mesh=plsc.VectorSubcoreMesh(core_axis_name="c", subcore_axis_name="s"),
    scratch_types=[
        pltpu.VMEM((HIST, BW), jnp.int32),
        pltpu.VMEM((BW, EMBEDDING_DIM), jnp.float32),
        pltpu.VMEM((BW, EMBEDDING_DIM), jnp.float32),
        pltpu.VMEM((EMBEDDING_DIM, BW), jnp.float32),
        pltpu.VMEM((EMBEDDING_DIM, BW), jnp.float32),
        pltpu.SemaphoreType.DMA,
        pltpu.SemaphoreType.DMA,
        pltpu.SemaphoreType.DMA,
        pltpu.SemaphoreType.DMA,
        pltpu.SemaphoreType.DMA,
    ],
    compiler_params=pltpu.CompilerParams(
        use_tc_tiling_on_sc=False, needs_layout_passes=False),
)


@jax.jit
def kernel(x, table):
    xt = x.T.astype(jnp.int32)
    out_t = _sc_gather(table, xt)
    return jnp.transpose(out_t, (2, 0, 1))
